# baseline (speedup 1.0000x reference)
import functools

import jax
import jax.numpy as jnp
from jax import lax
from jax.experimental import pallas as pl
from jax.experimental.pallas import tpu as pltpu
from jax.experimental.pallas import tpu_sc as plsc

SIZE = 256
THR = 0.01
NC, NS, L = 2, 16, 16
NW = NC * NS

C = 1024
GLEN = 512
TB = 65536
TBC = 4096


def _tc_idx_weights(N):
    nblk = N // TBC

    def body(c_ref, *outs):

        def axis(p):
            t = ((p + 1.0) * 256.0 - 1.0) / 2.0
            f = jnp.floor(t)
            i0 = f.astype(jnp.int32)
            w1 = t - f
            w0 = 1.0 - w1
            i1 = i0 + 1
            w0 = jnp.where(i0 >= 0, w0, 0.0)
            w1 = jnp.where(i1 <= SIZE - 1, w1, 0.0)
            i0c = jnp.maximum(i0, 0)
            i1c = jnp.minimum(i1, SIZE - 1)
            return i0, i0c, i1c, w0, w1

        c = c_ref[...]
        x0r, _, _, wx0, wx1 = axis(c[:, 0])
        _, y0, y1, wy0, wy1 = axis(c[:, 1])
        _, z0, z1, wz0, wz1 = axis(c[:, 2])
        kc = 0
        for zi, wz in ((z0, wz0), (z1, wz1)):
            for yi, wy in ((y0, wy0), (y1, wy1)):
                zy = zi * (SIZE * SIZE) + yi * SIZE
                wzy = wz * wy
                f = zy + x0r
                outs[kc][...] = jnp.maximum(f, 0)
                wlo = wzy * wx0
                whi = jnp.where(f >= 0, wzy * wx1, 0.0)
                wloq = (wlo * 32768.0 + 0.5).astype(jnp.int32)
                whiq = (whi * 32768.0 + 0.5).astype(jnp.int32)
                outs[4 + kc][...] = wloq | (whiq << 16)
                kc += 1

    return pl.pallas_call(
        body,
        grid=(nblk,),
        in_specs=[pl.BlockSpec((TBC, 3), lambda i: (i, 0))],
        out_specs=[pl.BlockSpec((TBC,), lambda i: (i,))] * 8,
        out_shape=[jax.ShapeDtypeStruct((N,), jnp.int32)] * 8,
    )


def _tc_table(v):
    nblk = v // TB

    def body(g_ref, gn_ref, o_ref):
        g = g_ref[...]
        q = (g * 32767.0 + 0.5).astype(jnp.int32)
        nxt = jnp.concatenate([g[1:], gn_ref[pl.ds(0, 1)]])
        qn = (nxt * 32767.0 + 0.5).astype(jnp.int32)
        o_ref[...] = q | (qn << 16)

    return pl.pallas_call(
        body,
        grid=(nblk,),
        in_specs=[pl.BlockSpec((TB,), lambda i: (i,)),
                  pl.BlockSpec((TB,), lambda i: (jnp.minimum(i + 1,
                                                             nblk - 1),))],
        out_specs=pl.BlockSpec((TB,), lambda i: (i,)),
        out_shape=jax.ShapeDtypeStruct((v,), jnp.int32),
    )


def _sc_gather_sum(N):
    PER_W = N // NW
    NCHUNK = PER_W // C
    NH = NCHUNK // 2
    G = C // L

    mesh = plsc.VectorSubcoreMesh(
        core_axis_name="c", subcore_axis_name="s",
        num_cores=NC, num_subcores=NS)

    buf_set = [
        pltpu.VMEM((4 * C,), jnp.int32),
        pltpu.VMEM((4 * C,), jnp.int32),
        pltpu.VMEM((4 * C,), jnp.int32),
        pltpu.VMEM((C,), jnp.int32),
        pltpu.SemaphoreType.DMA,
        pltpu.SemaphoreType.DMA,
        pltpu.SemaphoreType.DMA,
    ]

    @functools.partial(
        pl.kernel, mesh=mesh,
        out_type=jax.ShapeDtypeStruct((N,), jnp.int32),
        scratch_types=buf_set + buf_set,
    )
    def k(*args):
        idx_hbm = args[:4]
        w_hbm = args[4:8]
        tab_hbm = args[8]
        out_hbm = args[9]
        b0, b1 = args[10:17], args[17:24]
        wid = lax.axis_index("s") * NC + lax.axis_index("c")
        wbase = wid * PER_W
        ones = jnp.full((L,), 1, jnp.int32)
        lomask = jnp.full((L,), 32767, jnp.int32)
        wmask = jnp.full((L,), 65535, jnp.int32)
        thrq = jnp.full((L,), 41941, jnp.int32)

        def cbase(ci):
            return wbase + jnp.where(ci < NCHUNK, ci, 0) * C

        def start_in(ci, b):
            base = cbase(ci)
            idx_v, w_v, semin = b[0], b[1], b[4]
            for kc in range(4):
                pltpu.async_copy(idx_hbm[kc].at[pl.ds(base, C)],
                                 idx_v.at[pl.ds(kc * C, C)], semin)
            for kc in range(4):
                pltpu.async_copy(w_hbm[kc].at[pl.ds(base, C)],
                                 w_v.at[pl.ds(kc * C, C)], semin)

        def wait_in(b):
            idx_v, w_v, semin = b[0], b[1], b[4]
            for kc in range(4):
                pltpu.make_async_copy(idx_hbm[kc].at[pl.ds(0, C)],
                                      idx_v.at[pl.ds(kc * C, C)],
                                      semin).wait()
            for kc in range(4):
                pltpu.make_async_copy(w_hbm[kc].at[pl.ds(0, C)],
                                      w_v.at[pl.ds(kc * C, C)],
                                      semin).wait()

        def fire_gathers(b):
            idx_v, vals_v, semg = b[0], b[2], b[5]
            for o in range(0, 4 * C, GLEN):
                pltpu.async_copy(
                    tab_hbm.at[idx_v.at[pl.ds(o, GLEN)]],
                    vals_v.at[pl.ds(o, GLEN)], semg)

        def wait_gathers(b):
            idx_v, vals_v, semg = b[0], b[2], b[5]
            for o in range(0, 4 * C, GLEN):
                pltpu.make_async_copy(
                    tab_hbm.at[idx_v.at[pl.ds(o, GLEN)]],
                    vals_v.at[pl.ds(o, GLEN)], semg).wait()

        def pass2_out(ci, b, drain_prev):
            w_v, vals_v, out_v, semo = b[1], b[2], b[3], b[6]

            @pl.when(drain_prev)
            def _():
                pltpu.make_async_copy(
                    out_v, out_hbm.at[pl.ds(wbase, C)], semo).wait()

            def out_body(g, carry):
                off = g * L
                acc = None
                for kc in range(4):
                    pv = vals_v[pl.ds(kc * C + off, L)]
                    v0 = jnp.bitwise_and(pv, lomask)
                    v1 = lax.shift_right_logical(pv, 16)
                    pw = w_v[pl.ds(kc * C + off, L)]
                    wlo = jnp.bitwise_and(pw, wmask)
                    whi = lax.shift_right_logical(pw, 16)
                    c = (lax.shift_right_logical(v0 * wlo, 8)
                         + lax.shift_right_logical(v1 * whi, 8))
                    acc = c if acc is None else acc + c
                out_v[pl.ds(off, L)] = jnp.where(acc > thrq, ones, 0)
                return carry

            lax.fori_loop(0, G, out_body, 0)
            pltpu.async_copy(out_v, out_hbm.at[pl.ds(cbase(ci), C)], semo)

        start_in(0, b0)
        wait_in(b0)
        fire_gathers(b0)
        start_in(1, b1)

        def body(j, carry):
            wait_in(b1)
            fire_gathers(b1)
            wait_gathers(b0)
            pass2_out(2 * j, b0, j > 0)
            start_in(2 * j + 2, b0)

            wait_in(b0)
            fire_gathers(b0)
            wait_gathers(b1)
            pass2_out(2 * j + 1, b1, j > 0)
            start_in(2 * j + 3, b1)
            return carry

        lax.fori_loop(0, NH, body, 0)

        wait_gathers(b0)
        wait_in(b1)
        for b in (b0, b1):
            pltpu.make_async_copy(
                b[3], out_hbm.at[pl.ds(wbase, C)], b[6]).wait()

    return k


def kernel(coords, grid):
    n = coords.shape[0]
    gf = grid.reshape(-1)
    idx_w = _tc_idx_weights(n)(coords)
    out_i32 = _sc_gather_sum(n)(*idx_w, _tc_table(gf.shape[0])(gf, gf))
    return out_i32.astype(jnp.bool_)


# speedup vs baseline: 16.9484x; 16.9484x over previous
import functools

import jax
import jax.numpy as jnp
from jax import lax
from jax.experimental import pallas as pl
from jax.experimental.pallas import tpu as pltpu
from jax.experimental.pallas import tpu_sc as plsc

SIZE = 256
THR = 0.01
NC, NS, L = 2, 16, 16
NW = NC * NS

C = 1024
GLEN = 512
TB = 65536


def _tc_idx_weights(N):
    nblk = N // TB

    def body(xs_ref, ys_ref, zs_ref, *outs):

        def axis(p):
            t = ((p + 1.0) * 256.0 - 1.0) / 2.0
            f = jnp.floor(t)
            i0 = f.astype(jnp.int32)
            w1 = t - f
            w0 = 1.0 - w1
            i1 = i0 + 1
            w0 = jnp.where(i0 >= 0, w0, 0.0)
            w1 = jnp.where(i1 <= SIZE - 1, w1, 0.0)
            i0c = jnp.maximum(i0, 0)
            i1c = jnp.minimum(i1, SIZE - 1)
            return i0, i0c, i1c, w0, w1

        x0r, _, _, wx0, wx1 = axis(xs_ref[...])
        _, y0, y1, wy0, wy1 = axis(ys_ref[...])
        _, z0, z1, wz0, wz1 = axis(zs_ref[...])
        kc = 0
        for zi, wz in ((z0, wz0), (z1, wz1)):
            for yi, wy in ((y0, wy0), (y1, wy1)):
                zy = zi * (SIZE * SIZE) + yi * SIZE
                wzy = wz * wy
                f = zy + x0r
                outs[kc][...] = jnp.maximum(f, 0)
                wlo = wzy * wx0
                whi = jnp.where(f >= 0, wzy * wx1, 0.0)
                outs[4 + 2 * kc][...] = (wlo * 32768.0 + 0.5).astype(jnp.int32)
                outs[4 + 2 * kc + 1][...] = (whi * 32768.0
                                             + 0.5).astype(jnp.int32)
                kc += 1

    return pl.pallas_call(
        body,
        grid=(nblk,),
        in_specs=[pl.BlockSpec((TB,), lambda i: (i,))] * 3,
        out_specs=[pl.BlockSpec((TB,), lambda i: (i,))] * 12,
        out_shape=[jax.ShapeDtypeStruct((N,), jnp.int32)] * 12,
    )


def _tc_table(v):
    nblk = v // TB

    def body(g_ref, gn_ref, o_ref):
        g = g_ref[...]
        q = (g * 32767.0 + 0.5).astype(jnp.int32)
        nxt = jnp.concatenate([g[1:], gn_ref[pl.ds(0, 1)]])
        qn = (nxt * 32767.0 + 0.5).astype(jnp.int32)
        o_ref[...] = q | (qn << 16)

    return pl.pallas_call(
        body,
        grid=(nblk,),
        in_specs=[pl.BlockSpec((TB,), lambda i: (i,)),
                  pl.BlockSpec((TB,), lambda i: (jnp.minimum(i + 1,
                                                             nblk - 1),))],
        out_specs=pl.BlockSpec((TB,), lambda i: (i,)),
        out_shape=jax.ShapeDtypeStruct((v,), jnp.int32),
    )


def _sc_gather_sum(N):
    PER_W = N // NW
    NCHUNK = PER_W // C
    NH = NCHUNK // 2
    G = C // L

    mesh = plsc.VectorSubcoreMesh(
        core_axis_name="c", subcore_axis_name="s",
        num_cores=NC, num_subcores=NS)

    buf_set = [
        pltpu.VMEM((4 * C,), jnp.int32),
        pltpu.VMEM((8 * C,), jnp.int32),
        pltpu.VMEM((4 * C,), jnp.int32),
        pltpu.VMEM((C,), jnp.int32),
        pltpu.SemaphoreType.DMA,
        pltpu.SemaphoreType.DMA,
        pltpu.SemaphoreType.DMA,
    ]

    @functools.partial(
        pl.kernel, mesh=mesh,
        out_type=jax.ShapeDtypeStruct((N,), jnp.int32),
        scratch_types=buf_set + buf_set,
    )
    def k(*args):
        idx_hbm = args[:4]
        w_hbm = args[4:12]
        tab_hbm = args[12]
        out_hbm = args[13]
        b0, b1 = args[14:21], args[21:28]
        wid = lax.axis_index("s") * NC + lax.axis_index("c")
        wbase = wid * PER_W
        ones = jnp.full((L,), 1, jnp.int32)
        lomask = jnp.full((L,), 32767, jnp.int32)
        thrq = jnp.full((L,), 41941, jnp.int32)

        def cbase(ci):
            return wbase + jnp.where(ci < NCHUNK, ci, 0) * C

        def start_in(ci, b):
            base = cbase(ci)
            idx_v, w_v, semin = b[0], b[1], b[4]
            for kc in range(4):
                pltpu.async_copy(idx_hbm[kc].at[pl.ds(base, C)],
                                 idx_v.at[pl.ds(kc * C, C)], semin)
            for kc in range(8):
                pltpu.async_copy(w_hbm[kc].at[pl.ds(base, C)],
                                 w_v.at[pl.ds(kc * C, C)], semin)

        def wait_in(b):
            idx_v, w_v, semin = b[0], b[1], b[4]
            for kc in range(4):
                pltpu.make_async_copy(idx_hbm[kc].at[pl.ds(0, C)],
                                      idx_v.at[pl.ds(kc * C, C)],
                                      semin).wait()
            for kc in range(8):
                pltpu.make_async_copy(w_hbm[kc].at[pl.ds(0, C)],
                                      w_v.at[pl.ds(kc * C, C)],
                                      semin).wait()

        def fire_gathers(b):
            idx_v, vals_v, semg = b[0], b[2], b[5]
            for o in range(0, 4 * C, GLEN):
                pltpu.async_copy(
                    tab_hbm.at[idx_v.at[pl.ds(o, GLEN)]],
                    vals_v.at[pl.ds(o, GLEN)], semg)

        def wait_gathers(b):
            idx_v, vals_v, semg = b[0], b[2], b[5]
            for o in range(0, 4 * C, GLEN):
                pltpu.make_async_copy(
                    tab_hbm.at[idx_v.at[pl.ds(o, GLEN)]],
                    vals_v.at[pl.ds(o, GLEN)], semg).wait()

        def pass2_out(ci, b, drain_prev):
            w_v, vals_v, out_v, semo = b[1], b[2], b[3], b[6]

            @pl.when(drain_prev)
            def _():
                pltpu.make_async_copy(
                    out_v, out_hbm.at[pl.ds(wbase, C)], semo).wait()

            def out_body(g, carry):
                off = g * L
                acc = None
                for kc in range(4):
                    pv = vals_v[pl.ds(kc * C + off, L)]
                    v0 = jnp.bitwise_and(pv, lomask)
                    v1 = lax.shift_right_logical(pv, 16)
                    wlo = w_v[pl.ds((2 * kc) * C + off, L)]
                    whi = w_v[pl.ds((2 * kc + 1) * C + off, L)]
                    c = (lax.shift_right_logical(v0 * wlo, 8)
                         + lax.shift_right_logical(v1 * whi, 8))
                    acc = c if acc is None else acc + c
                out_v[pl.ds(off, L)] = jnp.where(acc > thrq, ones, 0)
                return carry

            lax.fori_loop(0, G, out_body, 0)
            pltpu.async_copy(out_v, out_hbm.at[pl.ds(cbase(ci), C)], semo)

        start_in(0, b0)
        wait_in(b0)
        fire_gathers(b0)
        start_in(1, b1)

        def body(j, carry):
            wait_in(b1)
            fire_gathers(b1)
            wait_gathers(b0)
            pass2_out(2 * j, b0, j > 0)
            start_in(2 * j + 2, b0)

            wait_in(b0)
            fire_gathers(b0)
            wait_gathers(b1)
            pass2_out(2 * j + 1, b1, j > 0)
            start_in(2 * j + 3, b1)
            return carry

        lax.fori_loop(0, NH, body, 0)

        wait_gathers(b0)
        wait_in(b1)
        for b in (b0, b1):
            pltpu.make_async_copy(
                b[3], out_hbm.at[pl.ds(wbase, C)], b[6]).wait()

    return k


def kernel(coords, grid):
    n = coords.shape[0]
    gf = grid.reshape(-1)
    idx_w = _tc_idx_weights(n)(coords[:, 0], coords[:, 1], coords[:, 2])
    out_i32 = _sc_gather_sum(n)(*idx_w, _tc_table(gf.shape[0])(gf, gf))
    return out_i32.astype(jnp.bool_)


# speedup vs baseline: 17.7266x; 1.0459x over previous
import functools

import jax
import jax.numpy as jnp
from jax import lax
from jax.experimental import pallas as pl
from jax.experimental.pallas import tpu as pltpu
from jax.experimental.pallas import tpu_sc as plsc

SIZE = 256
THR = 0.01
NC, NS, L = 2, 16, 16
NW = NC * NS

C = 1024
GLEN = 1024
TB = 65536


def _tc_idx_weights(N):
    nblk = N // TB

    def body(xs_ref, ys_ref, zs_ref, *outs):

        def axis(p):
            t = ((p + 1.0) * 256.0 - 1.0) / 2.0
            f = jnp.floor(t)
            i0 = f.astype(jnp.int32)
            w1 = t - f
            w0 = 1.0 - w1
            i1 = i0 + 1
            w0 = jnp.where(i0 >= 0, w0, 0.0)
            w1 = jnp.where(i1 <= SIZE - 1, w1, 0.0)
            i0c = jnp.maximum(i0, 0)
            i1c = jnp.minimum(i1, SIZE - 1)
            return i0, i0c, i1c, w0, w1

        x0r, _, _, wx0, wx1 = axis(xs_ref[...])
        _, y0, y1, wy0, wy1 = axis(ys_ref[...])
        _, z0, z1, wz0, wz1 = axis(zs_ref[...])
        kc = 0
        for zi, wz in ((z0, wz0), (z1, wz1)):
            for yi, wy in ((y0, wy0), (y1, wy1)):
                zy = zi * (SIZE * SIZE) + yi * SIZE
                wzy = wz * wy
                f = zy + x0r
                outs[kc][...] = jnp.maximum(f, 0)
                wlo = wzy * wx0
                whi = jnp.where(f >= 0, wzy * wx1, 0.0)
                outs[4 + 2 * kc][...] = (wlo * 32768.0 + 0.5).astype(jnp.int32)
                outs[4 + 2 * kc + 1][...] = (whi * 32768.0
                                             + 0.5).astype(jnp.int32)
                kc += 1

    return pl.pallas_call(
        body,
        grid=(nblk,),
        in_specs=[pl.BlockSpec((TB,), lambda i: (i,))] * 3,
        out_specs=[pl.BlockSpec((TB,), lambda i: (i,))] * 12,
        out_shape=[jax.ShapeDtypeStruct((N,), jnp.int32)] * 12,
    )


def _sc_gather_sum(N):
    PER_W = N // NW
    NCHUNK = PER_W // C
    NH = NCHUNK // 2
    G = C // L

    mesh = plsc.VectorSubcoreMesh(
        core_axis_name="c", subcore_axis_name="s",
        num_cores=NC, num_subcores=NS)

    buf_set = [
        pltpu.VMEM((4 * C,), jnp.int32),
        pltpu.VMEM((8 * C,), jnp.int32),
        pltpu.VMEM((4 * C,), jnp.int32),
        pltpu.VMEM((C,), jnp.int32),
        pltpu.SemaphoreType.DMA,
        pltpu.SemaphoreType.DMA,
        pltpu.SemaphoreType.DMA,
    ]

    @functools.partial(
        pl.kernel, mesh=mesh,
        out_type=jax.ShapeDtypeStruct((N,), jnp.int32),
        scratch_types=buf_set + buf_set,
    )
    def k(*args):
        idx_hbm = args[:4]
        w_hbm = args[4:12]
        tab_hbm = args[12]
        out_hbm = args[13]
        b0, b1 = args[14:21], args[21:28]
        wid = lax.axis_index("s") * NC + lax.axis_index("c")
        wbase = wid * PER_W
        ones = jnp.full((L,), 1, jnp.int32)
        lomask = jnp.full((L,), 32767, jnp.int32)
        thrq = jnp.full((L,), 41941, jnp.int32)

        def cbase(ci):
            return wbase + jnp.where(ci < NCHUNK, ci, 0) * C

        def start_in(ci, b):
            base = cbase(ci)
            idx_v, w_v, semin = b[0], b[1], b[4]
            for kc in range(4):
                pltpu.async_copy(idx_hbm[kc].at[pl.ds(base, C)],
                                 idx_v.at[pl.ds(kc * C, C)], semin)
            for kc in range(8):
                pltpu.async_copy(w_hbm[kc].at[pl.ds(base, C)],
                                 w_v.at[pl.ds(kc * C, C)], semin)

        def wait_in(b):
            idx_v, w_v, semin = b[0], b[1], b[4]
            for kc in range(4):
                pltpu.make_async_copy(idx_hbm[kc].at[pl.ds(0, C)],
                                      idx_v.at[pl.ds(kc * C, C)],
                                      semin).wait()
            for kc in range(8):
                pltpu.make_async_copy(w_hbm[kc].at[pl.ds(0, C)],
                                      w_v.at[pl.ds(kc * C, C)],
                                      semin).wait()

        def fire_gathers(b):
            idx_v, vals_v, semg = b[0], b[2], b[5]
            for o in range(0, 4 * C, GLEN):
                pltpu.async_copy(
                    tab_hbm.at[idx_v.at[pl.ds(o, GLEN)]],
                    vals_v.at[pl.ds(o, GLEN)], semg)

        def wait_gathers(b):
            idx_v, vals_v, semg = b[0], b[2], b[5]
            for o in range(0, 4 * C, GLEN):
                pltpu.make_async_copy(
                    tab_hbm.at[idx_v.at[pl.ds(o, GLEN)]],
                    vals_v.at[pl.ds(o, GLEN)], semg).wait()

        def pass2_out(ci, b, drain_prev):
            w_v, vals_v, out_v, semo = b[1], b[2], b[3], b[6]

            @pl.when(drain_prev)
            def _():
                pltpu.make_async_copy(
                    out_v, out_hbm.at[pl.ds(wbase, C)], semo).wait()

            def out_body(g, carry):
                off = g * L
                acc = None
                for kc in range(4):
                    pv = vals_v[pl.ds(kc * C + off, L)]
                    v0 = jnp.bitwise_and(pv, lomask)
                    v1 = lax.shift_right_logical(pv, 16)
                    wlo = w_v[pl.ds((2 * kc) * C + off, L)]
                    whi = w_v[pl.ds((2 * kc + 1) * C + off, L)]
                    c = (lax.shift_right_logical(v0 * wlo, 8)
                         + lax.shift_right_logical(v1 * whi, 8))
                    acc = c if acc is None else acc + c
                out_v[pl.ds(off, L)] = jnp.where(acc > thrq, ones, 0)
                return carry

            lax.fori_loop(0, G, out_body, 0)
            pltpu.async_copy(out_v, out_hbm.at[pl.ds(cbase(ci), C)], semo)

        start_in(0, b0)
        wait_in(b0)
        fire_gathers(b0)
        start_in(1, b1)

        def body(j, carry):
            wait_in(b1)
            fire_gathers(b1)
            wait_gathers(b0)
            pass2_out(2 * j, b0, j > 0)
            start_in(2 * j + 2, b0)

            wait_in(b0)
            fire_gathers(b0)
            wait_gathers(b1)
            pass2_out(2 * j + 1, b1, j > 0)
            start_in(2 * j + 3, b1)
            return carry

        lax.fori_loop(0, NH, body, 0)

        wait_gathers(b0)
        wait_in(b1)
        for b in (b0, b1):
            pltpu.make_async_copy(
                b[3], out_hbm.at[pl.ds(wbase, C)], b[6]).wait()

    return k


def _pair_table(grid):
    gf = grid.reshape(-1)
    q = (gf * 32767.0 + 0.5).astype(jnp.int32)
    hi = jnp.concatenate([q[1:], q[:1]])
    return q | (hi << 16)


def kernel(coords, grid):
    n = coords.shape[0]
    idx_w = _tc_idx_weights(n)(coords[:, 0], coords[:, 1], coords[:, 2])
    out_i32 = _sc_gather_sum(n)(*idx_w, _pair_table(grid))
    return out_i32.astype(jnp.bool_)


# speedup vs baseline: 17.7846x; 1.0033x over previous
import functools

import jax
import jax.numpy as jnp
from jax import lax
from jax.experimental import pallas as pl
from jax.experimental.pallas import tpu as pltpu
from jax.experimental.pallas import tpu_sc as plsc

SIZE = 256
THR = 0.01
NC, NS, L = 2, 16, 16
NW = NC * NS

C = 2048
GLEN = 512
TB = 65536


def _tc_idx_weights(N):
    nblk = N // TB

    def body(xs_ref, ys_ref, zs_ref, *outs):

        def axis(p):
            t = ((p + 1.0) * 256.0 - 1.0) / 2.0
            f = jnp.floor(t)
            i0 = f.astype(jnp.int32)
            w1 = t - f
            w0 = 1.0 - w1
            i1 = i0 + 1
            w0 = jnp.where(i0 >= 0, w0, 0.0)
            w1 = jnp.where(i1 <= SIZE - 1, w1, 0.0)
            i0c = jnp.maximum(i0, 0)
            i1c = jnp.minimum(i1, SIZE - 1)
            return i0, i0c, i1c, w0, w1

        x0r, _, _, wx0, wx1 = axis(xs_ref[...])
        _, y0, y1, wy0, wy1 = axis(ys_ref[...])
        _, z0, z1, wz0, wz1 = axis(zs_ref[...])
        kc = 0
        for zi, wz in ((z0, wz0), (z1, wz1)):
            for yi, wy in ((y0, wy0), (y1, wy1)):
                zy = zi * (SIZE * SIZE) + yi * SIZE
                wzy = wz * wy
                f = zy + x0r
                outs[kc][...] = jnp.maximum(f, 0)
                wlo = wzy * wx0
                whi = jnp.where(f >= 0, wzy * wx1, 0.0)
                outs[4 + 2 * kc][...] = (wlo * 32768.0 + 0.5).astype(jnp.int32)
                outs[4 + 2 * kc + 1][...] = (whi * 32768.0
                                             + 0.5).astype(jnp.int32)
                kc += 1

    return pl.pallas_call(
        body,
        grid=(nblk,),
        in_specs=[pl.BlockSpec((TB,), lambda i: (i,))] * 3,
        out_specs=[pl.BlockSpec((TB,), lambda i: (i,))] * 12,
        out_shape=[jax.ShapeDtypeStruct((N,), jnp.int32)] * 12,
    )


def _sc_gather_sum(N):
    PER_W = N // NW
    NCHUNK = PER_W // C
    NH = NCHUNK // 2
    G = C // L

    mesh = plsc.VectorSubcoreMesh(
        core_axis_name="c", subcore_axis_name="s",
        num_cores=NC, num_subcores=NS)

    buf_set = [
        pltpu.VMEM((4 * C,), jnp.int32),
        pltpu.VMEM((8 * C,), jnp.int32),
        pltpu.VMEM((4 * C,), jnp.int32),
        pltpu.VMEM((C,), jnp.int32),
        pltpu.SemaphoreType.DMA,
        pltpu.SemaphoreType.DMA,
        pltpu.SemaphoreType.DMA,
    ]

    @functools.partial(
        pl.kernel, mesh=mesh,
        out_type=jax.ShapeDtypeStruct((N,), jnp.int32),
        scratch_types=buf_set + buf_set,
    )
    def k(*args):
        idx_hbm = args[:4]
        w_hbm = args[4:12]
        tab_hbm = args[12]
        out_hbm = args[13]
        b0, b1 = args[14:21], args[21:28]
        wid = lax.axis_index("s") * NC + lax.axis_index("c")
        wbase = wid * PER_W
        ones = jnp.full((L,), 1, jnp.int32)
        lomask = jnp.full((L,), 32767, jnp.int32)
        thrq = jnp.full((L,), 41941, jnp.int32)

        def cbase(ci):
            return wbase + jnp.where(ci < NCHUNK, ci, 0) * C

        def start_in(ci, b):
            base = cbase(ci)
            idx_v, w_v, semin = b[0], b[1], b[4]
            for kc in range(4):
                pltpu.async_copy(idx_hbm[kc].at[pl.ds(base, C)],
                                 idx_v.at[pl.ds(kc * C, C)], semin)
            for kc in range(8):
                pltpu.async_copy(w_hbm[kc].at[pl.ds(base, C)],
                                 w_v.at[pl.ds(kc * C, C)], semin)

        def wait_in(b):
            idx_v, w_v, semin = b[0], b[1], b[4]
            for kc in range(4):
                pltpu.make_async_copy(idx_hbm[kc].at[pl.ds(0, C)],
                                      idx_v.at[pl.ds(kc * C, C)],
                                      semin).wait()
            for kc in range(8):
                pltpu.make_async_copy(w_hbm[kc].at[pl.ds(0, C)],
                                      w_v.at[pl.ds(kc * C, C)],
                                      semin).wait()

        def fire_gathers(b):
            idx_v, vals_v, semg = b[0], b[2], b[5]
            for o in range(0, 4 * C, GLEN):
                pltpu.async_copy(
                    tab_hbm.at[idx_v.at[pl.ds(o, GLEN)]],
                    vals_v.at[pl.ds(o, GLEN)], semg)

        def wait_gathers(b):
            idx_v, vals_v, semg = b[0], b[2], b[5]
            for o in range(0, 4 * C, GLEN):
                pltpu.make_async_copy(
                    tab_hbm.at[idx_v.at[pl.ds(o, GLEN)]],
                    vals_v.at[pl.ds(o, GLEN)], semg).wait()

        def pass2_out(ci, b, drain_prev):
            w_v, vals_v, out_v, semo = b[1], b[2], b[3], b[6]

            @pl.when(drain_prev)
            def _():
                pltpu.make_async_copy(
                    out_v, out_hbm.at[pl.ds(wbase, C)], semo).wait()

            def out_body(g, carry):
                off = g * L
                acc = None
                for kc in range(4):
                    pv = vals_v[pl.ds(kc * C + off, L)]
                    v0 = jnp.bitwise_and(pv, lomask)
                    v1 = lax.shift_right_logical(pv, 16)
                    wlo = w_v[pl.ds((2 * kc) * C + off, L)]
                    whi = w_v[pl.ds((2 * kc + 1) * C + off, L)]
                    c = (lax.shift_right_logical(v0 * wlo, 8)
                         + lax.shift_right_logical(v1 * whi, 8))
                    acc = c if acc is None else acc + c
                out_v[pl.ds(off, L)] = jnp.where(acc > thrq, ones, 0)
                return carry

            lax.fori_loop(0, G, out_body, 0)
            pltpu.async_copy(out_v, out_hbm.at[pl.ds(cbase(ci), C)], semo)

        start_in(0, b0)
        wait_in(b0)
        fire_gathers(b0)
        start_in(1, b1)

        def body(j, carry):
            wait_in(b1)
            fire_gathers(b1)
            wait_gathers(b0)
            pass2_out(2 * j, b0, j > 0)
            start_in(2 * j + 2, b0)

            wait_in(b0)
            fire_gathers(b0)
            wait_gathers(b1)
            pass2_out(2 * j + 1, b1, j > 0)
            start_in(2 * j + 3, b1)
            return carry

        lax.fori_loop(0, NH, body, 0)

        wait_gathers(b0)
        wait_in(b1)
        for b in (b0, b1):
            pltpu.make_async_copy(
                b[3], out_hbm.at[pl.ds(wbase, C)], b[6]).wait()

    return k


def _pair_table(grid):
    gf = grid.reshape(-1)
    q = (gf * 32767.0 + 0.5).astype(jnp.int32)
    hi = jnp.concatenate([q[1:], q[:1]])
    return q | (hi << 16)


def kernel(coords, grid):
    n = coords.shape[0]
    idx_w = _tc_idx_weights(n)(coords[:, 0], coords[:, 1], coords[:, 2])
    out_i32 = _sc_gather_sum(n)(*idx_w, _pair_table(grid))
    return out_i32.astype(jnp.bool_)
